# baseline (speedup 1.0000x reference)
import functools

import jax
import jax.numpy as jnp
from jax import lax
from jax.experimental import pallas as pl
from jax.experimental.pallas import tpu as pltpu
from jax.experimental.pallas import tpu_sc as plsc

DIM = 64
B = 16384


@functools.lru_cache(maxsize=None)
def _build_kernel():
    info = plsc.get_sparse_core_info()
    nc, ns = info.num_cores, info.num_subcores
    nw = nc * ns
    b_per_w = B // nw
    mesh = plsc.VectorSubcoreMesh(core_axis_name="c", subcore_axis_name="s")

    @functools.partial(
        pl.kernel,
        mesh=mesh,
        compiler_params=pltpu.CompilerParams(use_tc_tiling_on_sc=False),
        out_type=(
            jax.ShapeDtypeStruct((B, DIM), jnp.float32),
            jax.ShapeDtypeStruct((B, DIM), jnp.float32),
        ),
        scratch_types=[
            pltpu.VMEM((b_per_w,), jnp.int32),
            pltpu.VMEM((b_per_w,), jnp.int32),
            pltpu.VMEM((b_per_w, DIM), jnp.float32),
            pltpu.VMEM((b_per_w, DIM), jnp.float32),
            pltpu.SemaphoreType.DMA,
            pltpu.SemaphoreType.DMA,
        ],
    )
    def gather_kernel(user_hbm, item_hbm, ut_hbm, it_hbm, out_u, out_i,
                      idx_u, idx_i, rows_u, rows_i, sem_u, sem_i):
        wid = lax.axis_index("s") * nc + lax.axis_index("c")
        base = wid * b_per_w
        pltpu.sync_copy(user_hbm.at[pl.ds(base, b_per_w)], idx_u)
        pltpu.sync_copy(item_hbm.at[pl.ds(base, b_per_w)], idx_i)
        cu = pltpu.async_copy(ut_hbm.at[idx_u], rows_u, sem_u)
        ci = pltpu.async_copy(it_hbm.at[idx_i], rows_i, sem_i)
        cu.wait()
        pltpu.sync_copy(rows_u, out_u.at[pl.ds(base, b_per_w)])
        ci.wait()
        pltpu.sync_copy(rows_i, out_i.at[pl.ds(base, b_per_w)])

    return gather_kernel


def kernel(user, item, user_table, item_table):
    return _build_kernel()(user, item, user_table, item_table)


# speedup vs baseline: 1.4933x; 1.4933x over previous
import functools

import jax
import jax.numpy as jnp
from jax import lax
from jax.experimental import pallas as pl
from jax.experimental.pallas import tpu as pltpu
from jax.experimental.pallas import tpu_sc as plsc

DIM = 64
B = 16384
FIRE = 16


@functools.lru_cache(maxsize=None)
def _build_kernel():
    info = plsc.get_sparse_core_info()
    nc, ns = info.num_cores, info.num_subcores
    nw = nc * ns
    b_per_w = B // nw
    mesh = plsc.VectorSubcoreMesh(core_axis_name="c", subcore_axis_name="s")

    @functools.partial(
        pl.kernel,
        mesh=mesh,
        out_type=(
            jax.ShapeDtypeStruct((B, DIM), jnp.float32),
            jax.ShapeDtypeStruct((B, DIM), jnp.float32),
        ),
        scratch_types=[
            pltpu.VMEM((b_per_w,), jnp.int32),
            pltpu.VMEM((b_per_w,), jnp.int32),
            pltpu.VMEM((b_per_w // 2, DIM), jnp.float32),
            pltpu.VMEM((b_per_w // 2, DIM), jnp.float32),
            pltpu.SemaphoreType.DMA,
            pltpu.SemaphoreType.DMA,
        ],
    )
    def gather_kernel(user_hbm, item_hbm, ut_hbm, it_hbm, out_u, out_i,
                      idx_u, idx_i, rows_u, rows_i, sem_u, sem_i):
        wid = lax.axis_index("s") * nc + lax.axis_index("c")
        base = wid * b_per_w
        pltpu.sync_copy(user_hbm.at[pl.ds(base, b_per_w)], idx_u)
        pltpu.sync_copy(item_hbm.at[pl.ds(base, b_per_w)], idx_i)

        chunk = b_per_w // 2

        def gather_chunk(tbl_hbm, idx_v, rows_v, sem, c0):
            def wave(w, _):
                j0 = w * FIRE
                vec = idx_v[pl.ds(c0 + j0, FIRE)]
                copies = []
                for k in range(FIRE):
                    i = vec[k]
                    copies.append(
                        pltpu.async_copy(tbl_hbm.at[i], rows_v.at[j0 + k], sem)
                    )
                for c in copies:
                    c.wait()
                return ()

            lax.fori_loop(0, chunk // FIRE, wave, (), unroll=False)

        for tbl_hbm, idx_v, rows_v, sem, out in (
            (ut_hbm, idx_u, rows_u, sem_u, out_u),
            (it_hbm, idx_i, rows_i, sem_i, out_i),
        ):
            for c in range(2):
                gather_chunk(tbl_hbm, idx_v, rows_v, sem, c * chunk)
                pltpu.sync_copy(
                    rows_v, out.at[pl.ds(base + c * chunk, chunk)]
                )

    return gather_kernel


def kernel(user, item, user_table, item_table):
    return _build_kernel()(user, item, user_table, item_table)


# speedup vs baseline: 1.5717x; 1.0525x over previous
import functools

import jax
import jax.numpy as jnp
from jax import lax
from jax.experimental import pallas as pl
from jax.experimental.pallas import tpu as pltpu
from jax.experimental.pallas import tpu_sc as plsc

DIM = 64
B = 16384
WAVE = 16


@functools.lru_cache(maxsize=None)
def _build_kernel():
    info = plsc.get_sparse_core_info()
    nc, ns = info.num_cores, info.num_subcores
    nw = nc * ns
    b_per_w = B // nw
    chunk = b_per_w // 2
    mesh = plsc.VectorSubcoreMesh(core_axis_name="c", subcore_axis_name="s")

    @functools.partial(
        pl.kernel,
        mesh=mesh,
        out_type=(
            jax.ShapeDtypeStruct((B, DIM), jnp.float32),
            jax.ShapeDtypeStruct((B, DIM), jnp.float32),
        ),
        scratch_types=[
            pltpu.VMEM((b_per_w,), jnp.int32),
            pltpu.VMEM((b_per_w,), jnp.int32),
            pltpu.VMEM((chunk, DIM), jnp.float32),
            pltpu.VMEM((chunk, DIM), jnp.float32),
            pltpu.SemaphoreType.DMA,
            pltpu.SemaphoreType.DMA,
        ],
    )
    def gather_kernel(user_hbm, item_hbm, ut_hbm, it_hbm, out_u, out_i,
                      idx_u, idx_i, rows_u, rows_i, sem_u, sem_i):
        wid = lax.axis_index("s") * nc + lax.axis_index("c")
        base = wid * b_per_w
        pltpu.sync_copy(user_hbm.at[pl.ds(base, b_per_w)], idx_u)
        pltpu.sync_copy(item_hbm.at[pl.ds(base, b_per_w)], idx_i)

        def fire_chunk(tbl_hbm, idx_v, rows_v, sem, c0):
            def wave(w, _):
                j0 = w * WAVE
                vec = idx_v[pl.ds(c0 + j0, WAVE)]
                for k in range(WAVE):
                    pltpu.async_copy(tbl_hbm.at[vec[k]], rows_v.at[j0 + k],
                                     sem)
                return ()

            lax.fori_loop(0, chunk // WAVE, wave, (), unroll=False)

        def drain_chunk(tbl_hbm, rows_v, sem):
            pltpu.make_async_copy(
                tbl_hbm.at[pl.ds(0, chunk)], rows_v, sem
            ).wait()

        def write_chunk(rows_v, out, c0):
            pltpu.sync_copy(rows_v, out.at[pl.ds(base + c0, chunk)])

        fire_chunk(ut_hbm, idx_u, rows_u, sem_u, 0)
        fire_chunk(it_hbm, idx_i, rows_i, sem_i, 0)
        drain_chunk(ut_hbm, rows_u, sem_u)
        write_chunk(rows_u, out_u, 0)
        fire_chunk(ut_hbm, idx_u, rows_u, sem_u, chunk)
        drain_chunk(it_hbm, rows_i, sem_i)
        write_chunk(rows_i, out_i, 0)
        fire_chunk(it_hbm, idx_i, rows_i, sem_i, chunk)
        drain_chunk(ut_hbm, rows_u, sem_u)
        write_chunk(rows_u, out_u, chunk)
        drain_chunk(it_hbm, rows_i, sem_i)
        write_chunk(rows_i, out_i, chunk)

    return gather_kernel


def kernel(user, item, user_table, item_table):
    return _build_kernel()(user, item, user_table, item_table)


# speedup vs baseline: 1.5770x; 1.0034x over previous
import functools

import jax
import jax.numpy as jnp
from jax import lax
from jax.experimental import pallas as pl
from jax.experimental.pallas import tpu as pltpu
from jax.experimental.pallas import tpu_sc as plsc

DIM = 64
B = 16384
WAVE = 16


@functools.lru_cache(maxsize=None)
def _build_kernel():
    info = plsc.get_sparse_core_info()
    nc, ns = info.num_cores, info.num_subcores
    nw = nc * ns
    b_per_w = B // nw
    chunk = b_per_w // 2
    mesh = plsc.VectorSubcoreMesh(core_axis_name="c", subcore_axis_name="s")

    @functools.partial(
        pl.kernel,
        mesh=mesh,
        out_type=(
            jax.ShapeDtypeStruct((B, DIM), jnp.float32),
            jax.ShapeDtypeStruct((B, DIM), jnp.float32),
        ),
        scratch_types=[
            pltpu.VMEM((b_per_w,), jnp.int32),
            pltpu.VMEM((b_per_w,), jnp.int32),
            pltpu.VMEM((chunk, DIM), jnp.float32),
            pltpu.VMEM((chunk, DIM), jnp.float32),
            pltpu.SemaphoreType.DMA,
            pltpu.SemaphoreType.DMA,
        ],
    )
    def gather_kernel(user_hbm, item_hbm, ut_hbm, it_hbm, out_u, out_i,
                      idx_u, idx_i, rows_u, rows_i, sem_u, sem_i):
        wid = lax.axis_index("s") * nc + lax.axis_index("c")
        base = wid * b_per_w
        pltpu.sync_copy(user_hbm.at[pl.ds(base, b_per_w)], idx_u)
        pltpu.sync_copy(item_hbm.at[pl.ds(base, b_per_w)], idx_i)

        def fire_chunk(tbl_hbm, idx_v, rows_v, sem, c0):
            def wave(w, _):
                j0 = w * WAVE
                vec = idx_v[pl.ds(c0 + j0, WAVE)]
                for k in range(WAVE):
                    pltpu.async_copy(tbl_hbm.at[vec[k]], rows_v.at[j0 + k],
                                     sem)
                return ()

            lax.fori_loop(0, chunk // WAVE, wave, (), unroll=False)

        def drain_chunk(tbl_hbm, rows_v, sem):
            pltpu.make_async_copy(
                tbl_hbm.at[pl.ds(0, chunk)], rows_v, sem
            ).wait()

        def write_chunk(rows_v, out, c0):
            pltpu.sync_copy(rows_v, out.at[pl.ds(base + c0, chunk)])

        fire_chunk(ut_hbm, idx_u, rows_u, sem_u, 0)
        fire_chunk(it_hbm, idx_i, rows_i, sem_i, 0)
        drain_chunk(ut_hbm, rows_u, sem_u)
        write_chunk(rows_u, out_u, 0)
        fire_chunk(ut_hbm, idx_u, rows_u, sem_u, chunk)
        drain_chunk(it_hbm, rows_i, sem_i)
        write_chunk(rows_i, out_i, 0)
        fire_chunk(it_hbm, idx_i, rows_i, sem_i, chunk)
        drain_chunk(ut_hbm, rows_u, sem_u)
        write_chunk(rows_u, out_u, chunk)
        drain_chunk(it_hbm, rows_i, sem_i)
        write_chunk(rows_i, out_i, chunk)

    return gather_kernel


def kernel(user, item, user_table, item_table):
    ut_ref = jax.new_ref(user_table)
    it_ref = jax.new_ref(item_table)
    return _build_kernel()(user, item, ut_ref, it_ref)
